# Initial kernel scaffold; baseline (speedup 1.0000x reference)
import jax
import jax.numpy as jnp
from jax.experimental import pallas as pl


def kernel(x, h, c, edge_index, W_att, b_att, u_w, u_b, W_iou, U_iou, b_iou, U_f, W_f, b_f):
    raise NotImplementedError("write your pallas kernel here")


# speedup vs baseline: 3.5819x; 3.5819x over previous
import functools

import jax
import jax.numpy as jnp
from jax import lax
from jax.experimental import pallas as pl
from jax.experimental.pallas import tpu as pltpu
from jax.experimental.pallas import tpu_sc as plsc

N = 10000
E = 320000
H = 128
NP = 10240
NSC = 2
NSUB = 16
NW = NSC * NSUB
ET = E // NW
CH = 80
NCH = ET // CH
NQ = CH // 16
RT = NP // NSUB
BN = 400
GRID = N // BN

_mesh = plsc.VectorSubcoreMesh(core_axis_name="c", subcore_axis_name="s")
_dn = (((1,), (1,)), ((), ()))


def _pre_body(x_r, h_r, c_r, watt_r, batt_r, uw_r, ub_r, uf_r, wf_r, bf_r,
              wiou_r, biou_r, ta_o, tb_o, amax_o, hfc_o, xf_o, xiou_o):
    i = pl.program_id(0)
    xb = x_r[...]
    hb = h_r[...]
    cb = c_r[...]
    f32 = jnp.float32
    Wh = lax.dot_general(hb, watt_r[...], _dn, preferred_element_type=f32) + batt_r[...]
    Wx = lax.dot_general(xb, watt_r[...], _dn, preferred_element_type=f32) + batt_r[...]
    uw = uw_r[...]
    a = lax.dot_general(Wh, uw[:, :H], _dn, preferred_element_type=f32)
    bv = lax.dot_general(Wx, uw[:, H:], _dn, preferred_element_type=f32) + ub_r[...]
    ta_o[...] = a
    tb_o[...] = bv

    @pl.when(i == 0)
    def _():
        amax_o[...] = jnp.full((1, 16), -1e38, f32)

    amax_o[...] = jnp.maximum(amax_o[...], jnp.max(a))
    hfc_o[...] = jnp.concatenate(
        [lax.dot_general(hb, uf_r[...], _dn, preferred_element_type=f32), cb], axis=1)
    xf_o[...] = lax.dot_general(xb, wf_r[...], _dn, preferred_element_type=f32) + bf_r[...]
    xiou_o[...] = lax.dot_general(xb, wiou_r[...], _dn, preferred_element_type=f32) + biou_r[...]


def _full(shape):
    return pl.BlockSpec(shape, lambda i: (0,) * len(shape))


_pre = pl.pallas_call(
    _pre_body,
    grid=(GRID,),
    in_specs=[
        pl.BlockSpec((BN, H), lambda i: (i, 0)),
        pl.BlockSpec((BN, H), lambda i: (i, 0)),
        pl.BlockSpec((BN, H), lambda i: (i, 0)),
        _full((H, H)),
        _full((1, H)),
        _full((16, 2 * H)),
        _full((1, 16)),
        _full((H, H)),
        _full((H, H)),
        _full((1, H)),
        _full((3 * H, H)),
        _full((1, 3 * H)),
    ],
    out_specs=[
        pl.BlockSpec((BN, 16), lambda i: (i, 0)),
        pl.BlockSpec((BN, 16), lambda i: (i, 0)),
        pl.BlockSpec((1, 16), lambda i: (0, 0)),
        pl.BlockSpec((BN, 2 * H), lambda i: (i, 0)),
        pl.BlockSpec((BN, H), lambda i: (i, 0)),
        pl.BlockSpec((BN, 3 * H), lambda i: (i, 0)),
    ],
    out_shape=[
        jax.ShapeDtypeStruct((N, 16), jnp.float32),
        jax.ShapeDtypeStruct((N, 16), jnp.float32),
        jax.ShapeDtypeStruct((1, 16), jnp.float32),
        jax.ShapeDtypeStruct((N, 2 * H), jnp.float32),
        jax.ShapeDtypeStruct((N, H), jnp.float32),
        jax.ShapeDtypeStruct((N, 3 * H), jnp.float32),
    ],
)


def _k2_body(src2_h, dst2_h, h_h, a_h, bv_h, amax_h, zrt_h, znp_h,
             s_out, d_out,
             sidx_v, didx_v, srow_v, exbuf_v, amax_v,
             a_tab, bv_tab, den_tab, acc, sem1):
    cid = lax.axis_index("c")
    sid = lax.axis_index("s")
    wid = sid * NSC + cid
    rbase = pl.multiple_of(sid * RT, 8)
    pltpu.sync_copy(zrt_h, acc.at[pl.ds(rbase, RT)])
    pltpu.sync_copy(znp_h, den_tab)
    pltpu.sync_copy(a_h, a_tab)
    pltpu.sync_copy(bv_h, bv_tab)
    pltpu.sync_copy(amax_h, amax_v)
    plsc.subcore_barrier()

    def chunk(ci, carry):
        pltpu.sync_copy(src2_h.at[wid, ci], sidx_v)
        pltpu.sync_copy(dst2_h.at[wid, ci], didx_v)
        g1 = pltpu.async_copy(h_h.at[sidx_v], srow_v, sem1)
        am = amax_v[...]

        def qgrp(q, c2):
            sl = pl.ds(q * 16, 16)
            si = sidx_v[sl]
            di = didx_v[sl]
            av = plsc.load_gather(a_tab, [si])
            bvv = plsc.load_gather(bv_tab, [di])
            s = av + bvv
            s = jnp.where(s > 0.0, s, s * 0.01)
            M = am + bvv
            M = jnp.where(M > 0.0, M, M * 0.01)
            ex = jnp.exp(s - M)
            exbuf_v[sl] = ex
            plsc.addupdate_scatter(den_tab, [di], ex)
            return c2

        lax.fori_loop(0, NQ, qgrp, 0)
        g1.wait()

        def rowb(k, c2):
            exb = plsc.load_gather(exbuf_v, [jnp.full((16,), k, jnp.int32)])

            def colb(q, c3):
                sl = pl.ds(q * 16, 16)
                srow_v[k, sl] = srow_v[k, sl] * exb
                return c3

            lax.fori_loop(0, H // 16, colb, 0)
            return c2

        lax.fori_loop(0, CH, rowb, 0)
        pltpu.sync_copy(srow_v, acc.at[didx_v], add=True)
        return carry

    lax.fori_loop(0, NCH, chunk, 0)
    plsc.subcore_barrier()
    pltpu.sync_copy(acc.at[pl.ds(rbase, RT)], s_out.at[cid, pl.ds(rbase, RT)])
    pltpu.sync_copy(den_tab, d_out.at[cid, sid])


_k2 = functools.partial(
    pl.kernel,
    mesh=_mesh,
    out_type=[
        jax.ShapeDtypeStruct((NSC, NP, H), jnp.float32),
        jax.ShapeDtypeStruct((NSC, NSUB, NP), jnp.float32),
    ],
    scratch_types=[
        pltpu.VMEM((CH,), jnp.int32),
        pltpu.VMEM((CH,), jnp.int32),
        pltpu.VMEM((CH, H), jnp.float32),
        pltpu.VMEM((CH,), jnp.float32),
        pltpu.VMEM((16,), jnp.float32),
        pltpu.VMEM((NP,), jnp.float32),
        pltpu.VMEM((NP,), jnp.float32),
        pltpu.VMEM((NP,), jnp.float32),
        pltpu.VMEM_SHARED((NP, H), jnp.float32),
        pltpu.SemaphoreType.DMA,
    ],
    compiler_params=pltpu.CompilerParams(needs_layout_passes=False),
)(_k2_body)


def _k3_body(src2_h, dst2_h, hfc_h, xf_h, zrt_h, c_out,
             sidx_v, didx_v, hfc_v, xf_v, acc, sem1, sem2):
    cid = lax.axis_index("c")
    sid = lax.axis_index("s")
    wid = sid * NSC + cid
    rbase = pl.multiple_of(sid * RT, 8)
    pltpu.sync_copy(zrt_h, acc.at[pl.ds(rbase, RT)])
    plsc.subcore_barrier()

    def chunk(ci, carry):
        pltpu.sync_copy(src2_h.at[wid, ci], sidx_v)
        pltpu.sync_copy(dst2_h.at[wid, ci], didx_v)
        g1 = pltpu.async_copy(hfc_h.at[sidx_v], hfc_v, sem1)
        g2 = pltpu.async_copy(xf_h.at[didx_v], xf_v, sem2)
        g1.wait()
        g2.wait()

        def rowb(k, c2):
            def colb(q, c3):
                sl = pl.ds(q * 16, 16)
                z = hfc_v[k, sl] + xf_v[k, sl]
                f = 1.0 / (1.0 + jnp.exp(-z))
                xf_v[k, sl] = f * hfc_v[k, pl.ds(H + q * 16, 16)]
                return c3

            lax.fori_loop(0, H // 16, colb, 0)
            return c2

        lax.fori_loop(0, CH, rowb, 0)
        pltpu.sync_copy(xf_v, acc.at[didx_v], add=True)
        return carry

    lax.fori_loop(0, NCH, chunk, 0)
    plsc.subcore_barrier()
    pltpu.sync_copy(acc.at[pl.ds(rbase, RT)], c_out.at[cid, pl.ds(rbase, RT)])


_k3 = functools.partial(
    pl.kernel,
    mesh=_mesh,
    out_type=jax.ShapeDtypeStruct((NSC, NP, H), jnp.float32),
    scratch_types=[
        pltpu.VMEM((CH,), jnp.int32),
        pltpu.VMEM((CH,), jnp.int32),
        pltpu.VMEM((CH, 2 * H), jnp.float32),
        pltpu.VMEM((CH, H), jnp.float32),
        pltpu.VMEM_SHARED((NP, H), jnp.float32),
        pltpu.SemaphoreType.DMA,
        pltpu.SemaphoreType.DMA,
    ],
    compiler_params=pltpu.CompilerParams(needs_layout_passes=False),
)(_k3_body)


def _post_body(s_r, d_r, c_r, xiou_r, uiou_r, h_o, c_o):
    S = s_r[0] + s_r[1]
    Cg = c_r[0] + c_r[1]
    den = jnp.sum(d_r[...], axis=1)[:, None]
    ht = S / jnp.maximum(den, 1e-9)
    iou = lax.dot_general(ht, uiou_r[...], _dn,
                          preferred_element_type=jnp.float32) + xiou_r[...]
    ii = iou[:, :H]
    oo = iou[:, H:2 * H]
    uu = iou[:, 2 * H:]
    cn = jax.nn.sigmoid(ii) * jnp.tanh(uu) + Cg
    h_o[...] = jax.nn.sigmoid(oo) * jnp.tanh(cn)
    c_o[...] = cn


_post = pl.pallas_call(
    _post_body,
    grid=(GRID,),
    in_specs=[
        pl.BlockSpec((NSC, BN, H), lambda i: (0, i, 0)),
        pl.BlockSpec((BN, NW), lambda i: (i, 0)),
        pl.BlockSpec((NSC, BN, H), lambda i: (0, i, 0)),
        pl.BlockSpec((BN, 3 * H), lambda i: (i, 0)),
        _full((3 * H, H)),
    ],
    out_specs=[
        pl.BlockSpec((BN, H), lambda i: (i, 0)),
        pl.BlockSpec((BN, H), lambda i: (i, 0)),
    ],
    out_shape=[
        jax.ShapeDtypeStruct((N, H), jnp.float32),
        jax.ShapeDtypeStruct((N, H), jnp.float32),
    ],
)


def kernel(x, h, c, edge_index, W_att, b_att, u_w, u_b, W_iou, U_iou, b_iou,
           U_f, W_f, b_f):
    src = edge_index[0].astype(jnp.int32)
    dst = edge_index[1].astype(jnp.int32)
    ta, tb, amax, hfc, xf, xiou = _pre(
        x, h, c, W_att, b_att.reshape(1, H),
        jnp.tile(u_w.reshape(1, 2 * H), (16, 1)),
        jnp.tile(u_b.reshape(1, 1), (1, 16)),
        U_f, W_f, b_f, W_iou, b_iou)
    pad = jnp.zeros((NP - N,), jnp.float32)
    a_flat = jnp.concatenate([ta[:, 0], pad])
    bv_flat = jnp.concatenate([tb[:, 0], pad])
    zrt = jnp.zeros((RT, H), jnp.float32)
    znp = jnp.zeros((NP,), jnp.float32)
    src2 = src.reshape(NW, NCH, CH)
    dst2 = dst.reshape(NW, NCH, CH)
    spart, dpart = _k2(src2, dst2, h, a_flat, bv_flat, amax.reshape(16),
                       zrt, znp)
    cpart = _k3(src2, dst2, hfc, xf, zrt)
    dpart_t = dpart.reshape(NW, NP).T
    h_new, c_new = _post(spart, dpart_t, cpart, xiou, U_iou)
    return h_new, c_new
